# baseline (speedup 1.0000x reference)
import functools

import jax
import jax.numpy as jnp
import numpy as np
from jax import lax
from jax.experimental import pallas as pl
from jax.experimental.pallas import tpu as pltpu
from jax.experimental.pallas import tpu_sc as plsc

N = 10000
E = 320000
D = 128
H = 8
DH = 16
ROWS = 400
NBLK = N // ROWS

NC = 2
NS = 16
NW = NC * NS
HH = H // NC
DHH = HH * DH
EB = 40
EW = 20000
NB = EW // EB
N_ACC = 10240
RPT = N_ACC // NS
RCH = EB


def _qkv_body(h_ref, wq_ref, bq_ref, wk_ref, bk_ref, wv_ref, bv_ref,
              q_ref, k_ref, v_ref):
    hb = h_ref[...]
    q_ref[...] = (jnp.dot(hb, wq_ref[...], preferred_element_type=jnp.float32)
                  + bq_ref[...]) * 0.25
    k_ref[...] = jnp.dot(hb, wk_ref[...], preferred_element_type=jnp.float32) + bk_ref[...]
    v_ref[...] = jnp.dot(hb, wv_ref[...], preferred_element_type=jnp.float32) + bv_ref[...]


def _qkv(h, WQ_w, WQ_b, WK_w, WK_b, WV_w, WV_b):
    row_spec = pl.BlockSpec((ROWS, D), lambda i: (i, 0))
    w_spec = pl.BlockSpec((D, D), lambda i: (0, 0))
    b_spec = pl.BlockSpec((1, D), lambda i: (0, 0))
    out = jax.ShapeDtypeStruct((N, D), jnp.float32)
    return pl.pallas_call(
        _qkv_body,
        grid=(NBLK,),
        in_specs=[row_spec, w_spec, b_spec, w_spec, b_spec, w_spec, b_spec],
        out_specs=[row_spec, row_spec, row_spec],
        out_shape=[out, out, out],
    )(h, WQ_w.T, WQ_b.reshape(1, D), WK_w.T, WK_b.reshape(1, D),
      WV_w.T, WV_b.reshape(1, D))


def _ln_rows(x, g, b, eps=1e-5):
    mu = jnp.mean(x, axis=-1, keepdims=True)
    var = jnp.mean((x - mu) ** 2, axis=-1, keepdims=True)
    return (x - mu) * jax.lax.rsqrt(var + eps) * g + b


def _epi_body(acc_ref, z0_ref, z1_ref, zm_ref, h_ref, wo_ref, bo_ref,
              l1_ref, b1_ref, l2_ref, b2_ref,
              g1_ref, be1_ref, g2_ref, be2_ref, out_ref):
    zm = zm_ref[...]
    zv0 = jnp.dot(jnp.sum(z0_ref[...], axis=0, keepdims=True), zm,
                  preferred_element_type=jnp.float32)
    zv1 = jnp.dot(jnp.sum(z1_ref[...], axis=0, keepdims=True), zm,
                  preferred_element_type=jnp.float32)
    a0 = acc_ref[0] / zv0
    a1 = acc_ref[1] / zv1
    wo = wo_ref[...]
    h2 = (h_ref[...] + bo_ref[...]
          + jnp.dot(a0, wo[:DHH], preferred_element_type=jnp.float32)
          + jnp.dot(a1, wo[DHH:], preferred_element_type=jnp.float32))
    h2 = _ln_rows(h2, g1_ref[...], be1_ref[...])
    ff = jnp.dot(jax.nn.relu(
        jnp.dot(h2, l1_ref[...], preferred_element_type=jnp.float32) + b1_ref[...]),
        l2_ref[...], preferred_element_type=jnp.float32) + b2_ref[...]
    out_ref[...] = _ln_rows(h2 + ff, g2_ref[...], be2_ref[...])


_ZMASK = np.zeros((DH, DHH), np.float32)
for _h in range(HH):
    _ZMASK[_h, _h * DH:(_h + 1) * DH] = 1.0


def _epilogue(acc, zp0, zp1, h, WO_w, WO_b, l1_w, l1_b, l2_w, l2_b,
              ln1_g, ln1_b, ln2_g, ln2_b):
    row_spec = pl.BlockSpec((ROWS, D), lambda i: (i, 0))
    acc_spec = pl.BlockSpec((NC, ROWS, DHH), lambda i: (0, i, 0))
    w_spec = pl.BlockSpec((D, D), lambda i: (0, 0))
    b_spec = pl.BlockSpec((1, D), lambda i: (0, 0))
    z_spec = pl.BlockSpec((NS, DH), lambda i: (0, 0))
    zm_spec = pl.BlockSpec((DH, DHH), lambda i: (0, 0))
    return pl.pallas_call(
        _epi_body,
        grid=(NBLK,),
        in_specs=[acc_spec, z_spec, z_spec, zm_spec, row_spec, w_spec, b_spec,
                  w_spec, b_spec, w_spec, b_spec,
                  b_spec, b_spec, b_spec, b_spec],
        out_specs=row_spec,
        out_shape=jax.ShapeDtypeStruct((N, D), jnp.float32),
    )(acc, zp0, zp1, jnp.asarray(_ZMASK), h, WO_w.T, WO_b.reshape(1, D),
      l1_w.T, l1_b.reshape(1, D), l2_w.T, l2_b.reshape(1, D),
      ln1_g.reshape(1, D), ln1_b.reshape(1, D),
      ln2_g.reshape(1, D), ln2_b.reshape(1, D))


def _edge_sc_body(q0_hbm, q1_hbm, kv0_hbm, kv1_hbm,
                  src_hbm, dst_hbm, acc_out, z_out,
                  srcb_v, dstb_v, qrow_v, kvrow_v, msg_v, zacc_v,
                  acc_sh, semi, semg):
    c = lax.axis_index("c")
    s = lax.axis_index("s")
    wid = s * NC + c

    z16 = jnp.zeros((16,), jnp.float32)
    lanes = lax.iota(jnp.int32, 16)

    def _zero_row(i, _):
        for jj in range(DHH // 16):
            msg_v[0][i, pl.ds(jj * 16, 16)] = z16
        return 0
    lax.fori_loop(0, EB, _zero_row, 0)
    for kk in range(RPT // RCH):
        pltpu.sync_copy(msg_v[0].at[pl.ds(0, RCH)],
                        acc_sh.at[pl.ds(s * RPT + kk * RCH, RCH)])
    plsc.subcore_barrier()

    def _issue_idx(jb, b, r):
        pltpu.async_copy(src_hbm.at[s, jb], srcb_v[2 * b + r], semi[2 * b + r])
        pltpu.async_copy(dst_hbm.at[s, jb], dstb_v[2 * b + r], semi[2 * b + r])

    def _wait_idx(b, r):
        for _ in range(2):
            pltpu.make_async_copy(src_hbm.at[s, 0], srcb_v[2 * b + r],
                                  semi[2 * b + r]).wait()

    def _issue_gathers(b, r):
        @pl.when(c == 0)
        def _():
            pltpu.async_copy(q0_hbm.at[dstb_v[2 * b + r].at[0]], qrow_v[b], semg[b])
            pltpu.async_copy(kv0_hbm.at[srcb_v[2 * b + r].at[0]], kvrow_v[b], semg[b])

        @pl.when(c == 1)
        def _():
            pltpu.async_copy(q1_hbm.at[dstb_v[2 * b + r].at[0]], qrow_v[b], semg[b])
            pltpu.async_copy(kv1_hbm.at[srcb_v[2 * b + r].at[0]], kvrow_v[b], semg[b])

    def _wait_gathers(b):
        pltpu.make_async_copy(q0_hbm.at[pl.ds(0, EB)], qrow_v[b], semg[b]).wait()
        pltpu.make_async_copy(kv0_hbm.at[pl.ds(0, EB)], kvrow_v[b], semg[b]).wait()

    def _compute(j, b, zacc):
        ebase = s * EW + j * EB

        def _edge(e, zacc):
            s8 = z16
            for h in range(HH):
                qv = qrow_v[b][e, pl.ds(h * DH, DH)]
                kv = kvrow_v[b][e, pl.ds(h * DH, DH)]
                sh = jnp.sum(qv * kv)
                s8 = jnp.where(lanes == h, sh, s8)
            w8 = jnp.exp(jnp.clip(s8, -5.0, 5.0))
            w8 = jnp.where(ebase + e < E, w8, 0.0)
            zacc = zacc + w8
            for h in range(HH):
                wb = w8[h]
                vv = kvrow_v[b][e, pl.ds(DHH + h * DH, DH)]
                msg_v[b][e, pl.ds(h * DH, DH)] = wb * vv
            return zacc

        return lax.fori_loop(0, EB, _edge, zacc)

    for b in range(2):
        _issue_idx(b, b, 0)
    for b in range(2):
        _wait_idx(b, 0)
        _issue_gathers(b, 0)
    for b in range(2):
        _issue_idx(b + 2, b, 1)

    def _quad(t, zacc):
        for u in range(4):
            j = 4 * t + u
            b = u % 2
            r = u // 2
            _wait_gathers(b)
            zacc = _compute(j, b, zacc)
            pltpu.sync_copy(msg_v[b], acc_sh.at[dstb_v[2 * b + r].at[0]], add=True)
            _wait_idx(b, 1 - r)
            _issue_gathers(b, 1 - r)
            _issue_idx(j + 4, b, r)
        return zacc

    zacc = lax.fori_loop(0, NB // 4, _quad, z16)
    for b in range(2):
        _wait_gathers(b)
        _wait_idx(b, 1)
    plsc.subcore_barrier()

    zacc_v[0, :] = zacc
    pltpu.sync_copy(zacc_v, z_out.at[pl.ds(wid, 1)])
    for kk in range(RPT // RCH):
        rows = pl.ds(s * RPT + kk * RCH, RCH)
        pltpu.sync_copy(acc_sh.at[rows], msg_v[0].at[pl.ds(0, RCH)])
        pltpu.sync_copy(msg_v[0].at[pl.ds(0, RCH)], acc_out.at[c, rows])


def _edge_phase_sc(q, k, v, src, dst):
    zb = jnp.zeros((NS, 4, EB), jnp.int32)
    src_r = jnp.concatenate(
        [src.reshape(NS, NB, EB), zb], 1).reshape(NS, NB + 4, 1, EB)
    dst_r = jnp.concatenate(
        [dst.reshape(NS, NB, EB), zb], 1).reshape(NS, NB + 4, 1, EB)
    mesh = plsc.VectorSubcoreMesh(core_axis_name="c", subcore_axis_name="s", num_cores=NC)
    dbuf = lambda shape, dt: [pltpu.VMEM(shape, dt), pltpu.VMEM(shape, dt)]
    acc2, zpart = pl.kernel(
        _edge_sc_body,
        compiler_params=pltpu.CompilerParams(needs_layout_passes=False,
                                             use_tc_tiling_on_sc=False),
        out_type=[jax.ShapeDtypeStruct((NC, N_ACC, DHH), jnp.float32),
                  jax.ShapeDtypeStruct((NW, DH), jnp.float32)],
        mesh=mesh,
        scratch_types=[
            [pltpu.VMEM((1, EB), jnp.int32) for _ in range(4)],
            [pltpu.VMEM((1, EB), jnp.int32) for _ in range(4)],
            dbuf((EB, DHH), jnp.float32),
            dbuf((EB, D), jnp.float32),
            dbuf((EB, DHH), jnp.float32),
            pltpu.VMEM((1, DH), jnp.float32),
            pltpu.VMEM_SHARED((N_ACC, DHH), jnp.float32),
            [pltpu.SemaphoreType.DMA for _ in range(4)],
            [pltpu.SemaphoreType.DMA, pltpu.SemaphoreType.DMA],
        ],
    )(q[:, :DHH], q[:, DHH:],
      jnp.concatenate([k[:, :DHH], v[:, :DHH]], axis=1),
      jnp.concatenate([k[:, DHH:], v[:, DHH:]], axis=1),
      src_r, dst_r)
    zp = zpart.reshape(NS, NC, DH)
    return acc2, zp[:, 0, :], zp[:, 1, :]


def _edge_phase_xla(q, k, v, src, dst):
    qe = jnp.take(q, dst, axis=0).reshape(E, H, DH)
    ke = jnp.take(k, src, axis=0).reshape(E, H, DH)
    ve = jnp.take(v, src, axis=0).reshape(E, H, DH)
    s = (qe * ke).sum(axis=-1)
    w = jnp.exp(jnp.clip(s, -5.0, 5.0))
    z = w.sum(axis=0)
    msg = w[:, :, None] * ve
    acc = jnp.zeros((N, H, DH), jnp.float32).at[dst].add(msg)
    zpart = jnp.pad(z.reshape(1, H), ((0, 0), (0, D - H)))
    return acc.reshape(N, D), zpart


def kernel(edge_index, h, WQ_w, WQ_b, WK_w, WK_b, WV_w, WV_b, WO_w, WO_b,
           l1_w, l1_b, l2_w, l2_b, ln1_g, ln1_b, ln2_g, ln2_b):
    src = edge_index[0].astype(jnp.int32)
    dst = edge_index[1].astype(jnp.int32)
    q, k, v = _qkv(h, WQ_w, WQ_b, WK_w, WK_b, WV_w, WV_b)
    acc, zp0, zp1 = _edge_phase_sc(q, k, v, src, dst)
    return _epilogue(acc, zp0, zp1, h, WO_w, WO_b, l1_w, l1_b, l2_w, l2_b,
                     ln1_g, ln1_b, ln2_g, ln2_b)


# speedup vs baseline: 2.7229x; 2.7229x over previous
import functools

import jax
import jax.numpy as jnp
import numpy as np
from jax import lax
from jax.experimental import pallas as pl
from jax.experimental.pallas import tpu as pltpu
from jax.experimental.pallas import tpu_sc as plsc

N = 10000
E = 320000
D = 128
H = 8
DH = 16
ROWS = 400
NBLK = N // ROWS

NC = 2
NS = 16
NW = NC * NS
HH = H // NC
DHH = HH * DH
EB = 80
EW = 20000
NB = EW // EB
N_ACC = 10240
RPT = N_ACC // NS
RCH = EB


def _qkv_body(h_ref, wq_ref, bq_ref, wk_ref, bk_ref, wv_ref, bv_ref,
              q_ref, k_ref, v_ref):
    hb = h_ref[...]
    q_ref[...] = (jnp.dot(hb, wq_ref[...], preferred_element_type=jnp.float32)
                  + bq_ref[...]) * 0.25
    k_ref[...] = jnp.dot(hb, wk_ref[...], preferred_element_type=jnp.float32) + bk_ref[...]
    v_ref[...] = jnp.dot(hb, wv_ref[...], preferred_element_type=jnp.float32) + bv_ref[...]


def _qkv(h, WQ_w, WQ_b, WK_w, WK_b, WV_w, WV_b):
    row_spec = pl.BlockSpec((ROWS, D), lambda i: (i, 0))
    w_spec = pl.BlockSpec((D, D), lambda i: (0, 0))
    b_spec = pl.BlockSpec((1, D), lambda i: (0, 0))
    out = jax.ShapeDtypeStruct((N, D), jnp.float32)
    return pl.pallas_call(
        _qkv_body,
        grid=(NBLK,),
        in_specs=[row_spec, w_spec, b_spec, w_spec, b_spec, w_spec, b_spec],
        out_specs=[row_spec, row_spec, row_spec],
        out_shape=[out, out, out],
    )(h, WQ_w.T, WQ_b.reshape(1, D), WK_w.T, WK_b.reshape(1, D),
      WV_w.T, WV_b.reshape(1, D))


def _ln_rows(x, g, b, eps=1e-5):
    mu = jnp.mean(x, axis=-1, keepdims=True)
    var = jnp.mean((x - mu) ** 2, axis=-1, keepdims=True)
    return (x - mu) * jax.lax.rsqrt(var + eps) * g + b


def _epi_body(acc_ref, z0_ref, z1_ref, zm_ref, h_ref, wo_ref, bo_ref,
              l1_ref, b1_ref, l2_ref, b2_ref,
              g1_ref, be1_ref, g2_ref, be2_ref, out_ref):
    zm = zm_ref[...]
    zv0 = jnp.dot(jnp.sum(z0_ref[...], axis=0, keepdims=True), zm,
                  preferred_element_type=jnp.float32)
    zv1 = jnp.dot(jnp.sum(z1_ref[...], axis=0, keepdims=True), zm,
                  preferred_element_type=jnp.float32)
    a0 = acc_ref[0] / zv0
    a1 = acc_ref[1] / zv1
    wo = wo_ref[...]
    h2 = (h_ref[...] + bo_ref[...]
          + jnp.dot(a0, wo[:DHH], preferred_element_type=jnp.float32)
          + jnp.dot(a1, wo[DHH:], preferred_element_type=jnp.float32))
    h2 = _ln_rows(h2, g1_ref[...], be1_ref[...])
    ff = jnp.dot(jax.nn.relu(
        jnp.dot(h2, l1_ref[...], preferred_element_type=jnp.float32) + b1_ref[...]),
        l2_ref[...], preferred_element_type=jnp.float32) + b2_ref[...]
    out_ref[...] = _ln_rows(h2 + ff, g2_ref[...], be2_ref[...])


_ZMASK = np.zeros((DH, DHH), np.float32)
for _h in range(HH):
    _ZMASK[_h, _h * DH:(_h + 1) * DH] = 1.0


def _epilogue(acc, zp0, zp1, h, WO_w, WO_b, l1_w, l1_b, l2_w, l2_b,
              ln1_g, ln1_b, ln2_g, ln2_b):
    row_spec = pl.BlockSpec((ROWS, D), lambda i: (i, 0))
    acc_spec = pl.BlockSpec((NC, ROWS, DHH), lambda i: (0, i, 0))
    w_spec = pl.BlockSpec((D, D), lambda i: (0, 0))
    b_spec = pl.BlockSpec((1, D), lambda i: (0, 0))
    z_spec = pl.BlockSpec((NS, DH), lambda i: (0, 0))
    zm_spec = pl.BlockSpec((DH, DHH), lambda i: (0, 0))
    return pl.pallas_call(
        _epi_body,
        grid=(NBLK,),
        in_specs=[acc_spec, z_spec, z_spec, zm_spec, row_spec, w_spec, b_spec,
                  w_spec, b_spec, w_spec, b_spec,
                  b_spec, b_spec, b_spec, b_spec],
        out_specs=row_spec,
        out_shape=jax.ShapeDtypeStruct((N, D), jnp.float32),
    )(acc, zp0, zp1, jnp.asarray(_ZMASK), h, WO_w.T, WO_b.reshape(1, D),
      l1_w.T, l1_b.reshape(1, D), l2_w.T, l2_b.reshape(1, D),
      ln1_g.reshape(1, D), ln1_b.reshape(1, D),
      ln2_g.reshape(1, D), ln2_b.reshape(1, D))


def _edge_sc_body(q0_hbm, q1_hbm, k0_hbm, k1_hbm, v0_hbm, v1_hbm,
                  src_hbm, dst_hbm, acc_out, z_out,
                  srcb_v, dstb_v, qrow_v, krow_v, vrow_v, msg_v, zacc_v,
                  acc_sh, semi, semg):
    c = lax.axis_index("c")
    s = lax.axis_index("s")
    wid = s * NC + c

    z16 = jnp.zeros((16,), jnp.float32)
    lanes = lax.iota(jnp.int32, 16)

    def _zero_row(i, _):
        for jj in range(DHH // 16):
            msg_v[0][i, pl.ds(jj * 16, 16)] = z16
        return 0
    lax.fori_loop(0, EB, _zero_row, 0)
    for kk in range(RPT // RCH):
        pltpu.sync_copy(msg_v[0].at[pl.ds(0, RCH)],
                        acc_sh.at[pl.ds(s * RPT + kk * RCH, RCH)])
    plsc.subcore_barrier()

    def _issue_idx(jb, b, r):
        pltpu.async_copy(src_hbm.at[s, jb], srcb_v[2 * b + r], semi[2 * b + r])
        pltpu.async_copy(dst_hbm.at[s, jb], dstb_v[2 * b + r], semi[2 * b + r])

    def _wait_idx(b, r):
        for _ in range(2):
            pltpu.make_async_copy(src_hbm.at[s, 0], srcb_v[2 * b + r],
                                  semi[2 * b + r]).wait()

    def _issue_gathers(b, r):
        @pl.when(c == 0)
        def _():
            pltpu.async_copy(q0_hbm.at[dstb_v[2 * b + r].at[0]], qrow_v[b], semg[b])
            pltpu.async_copy(k0_hbm.at[srcb_v[2 * b + r].at[0]], krow_v[b], semg[b])
            pltpu.async_copy(v0_hbm.at[srcb_v[2 * b + r].at[0]], vrow_v[b], semg[b])

        @pl.when(c == 1)
        def _():
            pltpu.async_copy(q1_hbm.at[dstb_v[2 * b + r].at[0]], qrow_v[b], semg[b])
            pltpu.async_copy(k1_hbm.at[srcb_v[2 * b + r].at[0]], krow_v[b], semg[b])
            pltpu.async_copy(v1_hbm.at[srcb_v[2 * b + r].at[0]], vrow_v[b], semg[b])

    def _wait_gathers(b):
        for _ in range(3):
            pltpu.make_async_copy(q0_hbm.at[pl.ds(0, EB)], qrow_v[b], semg[b]).wait()

    def _compute(j, b, zacc):
        ebase = s * EW + j * EB

        def _edge(e, zacc):
            s8 = z16
            for h in range(HH):
                qv = qrow_v[b][e, pl.ds(h * DH, DH)]
                kv = krow_v[b][e, pl.ds(h * DH, DH)]
                sh = jnp.sum(qv * kv)
                s8 = jnp.where(lanes == h, sh, s8)
            w8 = jnp.exp(jnp.clip(s8, -5.0, 5.0))
            w8 = jnp.where(ebase + e < E, w8, 0.0)
            zacc = zacc + w8
            for h in range(HH):
                wb = w8[h]
                vv = vrow_v[b][e, pl.ds(h * DH, DH)]
                msg_v[b][e, pl.ds(h * DH, DH)] = wb * vv
            return zacc

        return lax.fori_loop(0, EB, _edge, zacc)

    for b in range(2):
        _issue_idx(b, b, 0)
    for b in range(2):
        _wait_idx(b, 0)
        _issue_gathers(b, 0)
    for b in range(2):
        _issue_idx(b + 2, b, 1)

    def _quad(t, zacc):
        for u in range(4):
            j = 4 * t + u
            b = u % 2
            r = u // 2
            _wait_gathers(b)
            zacc = _compute(j, b, zacc)
            pltpu.sync_copy(msg_v[b], acc_sh.at[dstb_v[2 * b + r].at[0]], add=True)
            _wait_idx(b, 1 - r)
            _issue_gathers(b, 1 - r)
            _issue_idx(j + 4, b, r)
        return zacc

    zacc = lax.fori_loop(0, NB // 4, _quad, z16)
    for b in range(2):
        _wait_gathers(b)
        _wait_idx(b, 1)
    plsc.subcore_barrier()

    zacc_v[0, :] = zacc
    pltpu.sync_copy(zacc_v, z_out.at[pl.ds(wid, 1)])
    for kk in range(RPT // RCH):
        rows = pl.ds(s * RPT + kk * RCH, RCH)
        pltpu.sync_copy(acc_sh.at[rows], msg_v[0].at[pl.ds(0, RCH)])
        pltpu.sync_copy(msg_v[0].at[pl.ds(0, RCH)], acc_out.at[c, rows])


def _edge_phase_sc(q, k, v, src, dst):
    zb = jnp.zeros((NS, 4, EB), jnp.int32)
    src_r = jnp.concatenate(
        [src.reshape(NS, NB, EB), zb], 1).reshape(NS, NB + 4, 1, EB)
    dst_r = jnp.concatenate(
        [dst.reshape(NS, NB, EB), zb], 1).reshape(NS, NB + 4, 1, EB)
    mesh = plsc.VectorSubcoreMesh(core_axis_name="c", subcore_axis_name="s", num_cores=NC)
    dbuf = lambda shape, dt: [pltpu.VMEM(shape, dt), pltpu.VMEM(shape, dt)]
    acc2, zpart = pl.kernel(
        _edge_sc_body,
        compiler_params=pltpu.CompilerParams(needs_layout_passes=False,
                                             use_tc_tiling_on_sc=False),
        out_type=[jax.ShapeDtypeStruct((NC, N_ACC, DHH), jnp.float32),
                  jax.ShapeDtypeStruct((NW, DH), jnp.float32)],
        mesh=mesh,
        scratch_types=[
            [pltpu.VMEM((1, EB), jnp.int32) for _ in range(4)],
            [pltpu.VMEM((1, EB), jnp.int32) for _ in range(4)],
            dbuf((EB, DHH), jnp.float32),
            dbuf((EB, DHH), jnp.float32),
            dbuf((EB, DHH), jnp.float32),
            dbuf((EB, DHH), jnp.float32),
            pltpu.VMEM((1, DH), jnp.float32),
            pltpu.VMEM_SHARED((N_ACC, DHH), jnp.float32),
            [pltpu.SemaphoreType.DMA for _ in range(4)],
            [pltpu.SemaphoreType.DMA, pltpu.SemaphoreType.DMA],
        ],
    )(q[:, :DHH], q[:, DHH:], k[:, :DHH], k[:, DHH:], v[:, :DHH], v[:, DHH:],
      src_r, dst_r)
    zp = zpart.reshape(NS, NC, DH)
    return acc2, zp[:, 0, :], zp[:, 1, :]


def _edge_phase_xla(q, k, v, src, dst):
    qe = jnp.take(q, dst, axis=0).reshape(E, H, DH)
    ke = jnp.take(k, src, axis=0).reshape(E, H, DH)
    ve = jnp.take(v, src, axis=0).reshape(E, H, DH)
    s = (qe * ke).sum(axis=-1)
    w = jnp.exp(jnp.clip(s, -5.0, 5.0))
    z = w.sum(axis=0)
    msg = w[:, :, None] * ve
    acc = jnp.zeros((N, H, DH), jnp.float32).at[dst].add(msg)
    zpart = jnp.pad(z.reshape(1, H), ((0, 0), (0, D - H)))
    return acc.reshape(N, D), zpart


def kernel(edge_index, h, WQ_w, WQ_b, WK_w, WK_b, WV_w, WV_b, WO_w, WO_b,
           l1_w, l1_b, l2_w, l2_b, ln1_g, ln1_b, ln2_g, ln2_b):
    src = edge_index[0].astype(jnp.int32)
    dst = edge_index[1].astype(jnp.int32)
    q, k, v = _qkv(h, WQ_w, WQ_b, WK_w, WK_b, WV_w, WV_b)
    acc, zp0, zp1 = _edge_phase_sc(q, k, v, src, dst)
    return _epilogue(acc, zp0, zp1, h, WO_w, WO_b, l1_w, l1_b, l2_w, l2_b,
                     ln1_g, ln1_b, ln2_g, ln2_b)
